# baseline (speedup 1.0000x reference)
import functools
import math

import jax
import jax.numpy as jnp
from jax import lax
from jax.experimental import pallas as pl
from jax.experimental.pallas import tpu as pltpu
from jax.experimental.pallas import tpu_sc as plsc

NC = 2
NS = 16
NW = NC * NS
LANES = 16

D = 64
BATCH = 4096
HIST = 200
NB = BATCH // 128
ND = D // 8
SCALE = math.sqrt(float(D))

_mesh = plsc.VectorSubcoreMesh(
    core_axis_name="c", subcore_axis_name="s", num_cores=NC, num_subcores=NS
)


@functools.partial(
    pl.kernel,
    out_type=jax.ShapeDtypeStruct((HIST, ND, NB, 8, 128), jnp.float32),
    mesh=_mesh,
    scratch_types=[
        pltpu.VMEM((128,), jnp.int32),
        pltpu.VMEM((128, D), jnp.float32),
        pltpu.VMEM((8, 128), jnp.float32),
        pltpu.SemaphoreType.DMA,
    ],
    compiler_params=pltpu.CompilerParams(
        use_tc_tiling_on_sc=False, needs_layout_passes=False
    ),
)
def _emb_lookup(table_hbm, srct_hbm, out_hbm, idx_v, rows_v, tile_v, sem):
    wid = lax.axis_index("s") * NC + lax.axis_index("c")

    iotas = [lax.iota(jnp.int32, LANES) + (l0 * LANES) for l0 in range(8)]

    @pl.loop(0, HIST)
    def _step(h):
        pltpu.sync_copy(srct_hbm.at[h, pl.ds(wid * 128, 128)], idx_v)
        pltpu.async_copy(table_hbm.at[idx_v], rows_v, sem).wait()
        for td in range(ND):
            for s in range(8):
                col = jnp.full((LANES,), td * 8 + s, jnp.int32)
                for l0 in range(8):
                    v = plsc.load_gather(rows_v, [iotas[l0], col])
                    tile_v[s, pl.ds(l0 * LANES, LANES)] = v * SCALE
            pltpu.sync_copy(tile_v, out_hbm.at[h, td, wid])


def kernel(src, emb_weight):
    src_t = src.T.astype(jnp.int32)
    x = _emb_lookup(emb_weight, src_t)
    out = jnp.transpose(x, (2, 4, 0, 1, 3))
    return out.reshape(BATCH, HIST, D)
